# baseline (speedup 1.0000x reference)
import jax
import jax.numpy as jnp
from jax.experimental import pallas as pl
from jax.experimental.pallas import tpu as pltpu
from jax.experimental.pallas import tpu_sc as plsc

NUM_TOKENS = 4096
HIDDEN = 1024
FFN = 4096
NUM_EXPERTS = 8
TOP_K = 2

TB = 256
BLK = 256
NBLK = (NUM_TOKENS * TOP_K) // BLK + NUM_EXPERTS
CAP = NBLK * BLK
CHUNK = 512
NCH = NUM_TOKENS // CHUNK
SCW = 16
HU = HIDDEN // 2


def _router_kernel(x_ref, gw_ref, logits_ref, am_ref, bm_ref, pvec_ref):
    x = x_ref[...]
    gw = gw_ref[...]
    logits = jax.lax.dot_general(
        x, gw, (((1,), (1,)), ((), ())),
        preferred_element_type=jnp.float32,
        precision=jax.lax.Precision.DEFAULT)
    logits_ref[...] = logits
    e = gw.shape[0]
    iota = jax.lax.broadcasted_iota(jnp.int32, logits.shape, 1)
    m1 = jnp.max(logits, axis=1, keepdims=True)
    i1 = jnp.min(jnp.where(logits == m1, iota, e), axis=1, keepdims=True)
    a = iota == i1
    masked = jnp.where(a, -jnp.inf, logits)
    m2 = jnp.max(masked, axis=1, keepdims=True)
    i2 = jnp.min(jnp.where(masked == m2, iota, e), axis=1, keepdims=True)
    b = iota == i2
    q = jnp.exp(m2 - m1)
    p0 = 1.0 / (1.0 + q)
    p1 = q / (1.0 + q)
    af = a.astype(jnp.float32)
    bf = b.astype(jnp.float32)
    am_ref[...] = af
    bm_ref[...] = bf
    pvec_ref[...] = af * p0 + bf * p1


def _router(hidden_states, gate_w):
    t, _ = hidden_states.shape
    e = gate_w.shape[0]
    out_shapes = tuple(
        jax.ShapeDtypeStruct((t, e), jnp.float32) for _ in range(4))
    small = pl.BlockSpec((TB, e), lambda i: (i, 0))
    return pl.pallas_call(
        _router_kernel,
        grid=(t // TB,),
        in_specs=[
            pl.BlockSpec((TB, HIDDEN), lambda i: (i, 0)),
            pl.BlockSpec((e, HIDDEN), lambda i: (0, 0)),
        ],
        out_specs=(small, small, small, small),
        out_shape=out_shapes,
        compiler_params=pltpu.CompilerParams(
            dimension_semantics=("parallel",)),
    )(hidden_states, gate_w)


def _plan_kernel(am_ref, bm_ref, pos0_ref, pos1_ref, meta_ref):
    am = am_ref[...]
    bm = bm_ref[...]
    s = am + bm
    ri = jax.lax.broadcasted_iota(jnp.int32, (CHUNK, CHUNK), 0)
    ci = jax.lax.broadcasted_iota(jnp.int32, (CHUNK, CHUNK), 1)
    lstrict = (ci < ri).astype(jnp.float32)
    off = jnp.zeros((1, NUM_EXPERTS), jnp.float32)
    pieces = []
    for c in range(NCH):
        sc = s[c * CHUNK:(c + 1) * CHUNK]
        p = jax.lax.dot_general(
            lstrict, sc, (((1,), (0,)), ((), ())),
            preferred_element_type=jnp.float32)
        pieces.append(p + off)
        off = off + jnp.sum(sc, axis=0, keepdims=True)
    prefix = jnp.concatenate(pieces, axis=0)
    counts = off
    nb = jnp.floor((counts + (BLK - 1.0)) * (1.0 / BLK))
    ei = jax.lax.broadcasted_iota(jnp.int32, (NUM_EXPERTS, NUM_EXPERTS), 0)
    ej = jax.lax.broadcasted_iota(jnp.int32, (NUM_EXPERTS, NUM_EXPERTS), 1)
    uincl = (ei <= ej).astype(jnp.float32)
    rbend = jax.lax.dot_general(
        nb, uincl, (((1,), (0,)), ((), ())),
        preferred_element_type=jnp.float32)
    base = float(BLK) * (rbend - nb)
    tgt = base + prefix
    pos0_ref[...] = jnp.sum(am * tgt, axis=1, keepdims=True).astype(jnp.int32)
    pos1_ref[...] = jnp.sum(bm * tgt, axis=1, keepdims=True).astype(jnp.int32)
    meta = jnp.concatenate(
        [jnp.zeros((1, 1), jnp.float32), rbend,
         jnp.zeros((1, 16 - NUM_EXPERTS - 1), jnp.float32)], axis=1)
    meta_ref[...] = meta.astype(jnp.int32)


def _plan(am, bm):
    t, e = am.shape
    return pl.pallas_call(
        _plan_kernel,
        grid=(1,),
        in_specs=[
            pl.BlockSpec((t, e), lambda i: (0, 0)),
            pl.BlockSpec((t, e), lambda i: (0, 0)),
        ],
        out_specs=(
            pl.BlockSpec((t, 1), lambda i: (0, 0)),
            pl.BlockSpec((t, 1), lambda i: (0, 0)),
            pl.BlockSpec((1, 16), lambda i: (0, 0)),
        ),
        out_shape=(
            jax.ShapeDtypeStruct((t, 1), jnp.int32),
            jax.ShapeDtypeStruct((t, 1), jnp.int32),
            jax.ShapeDtypeStruct((1, 16), jnp.int32),
        ),
    )(am, bm)


def _sc_mesh():
    return plsc.VectorSubcoreMesh(core_axis_name="c", subcore_axis_name="s")


def _sc_dispatch(xb, p0_2d, p1_2d):
    @pl.kernel(out_type=jax.ShapeDtypeStruct((CAP, HIDDEN), jnp.float32),
               mesh=_sc_mesh())
    def k(x_hbm, p0_hbm, p1_hbm, o_hbm):
        def body(x_vmem, i_vmem):
            pltpu.sync_copy(x_vmem, o_hbm.at[i_vmem.at[0]])
        for p_hbm in (p0_hbm, p1_hbm):
            pltpu.emit_pipeline(
                body,
                grid=(NUM_TOKENS // SCW,),
                in_specs=[
                    pl.BlockSpec((SCW, HIDDEN), lambda i: (i, 0)),
                    pl.BlockSpec((1, SCW), lambda i: (i, 0)),
                ],
                out_specs=[],
                core_axis_name=("c", "s"),
                dimension_semantics=(pltpu.PARALLEL,),
            )(x_hbm, p_hbm)
    return k(xb, p0_2d, p1_2d)


def _sc_collect(y_packed, p0_2d, p1_2d):
    out_types = [jax.ShapeDtypeStruct((NUM_TOKENS, HU), jnp.uint32)] * 2
    @pl.kernel(out_type=out_types, mesh=_sc_mesh())
    def k(y_hbm, p0_hbm, p1_hbm, o0_hbm, o1_hbm):
        def body(i_vmem, o_vmem):
            pltpu.sync_copy(y_hbm.at[i_vmem.at[0]], o_vmem)
        for p_hbm, o_hbm in ((p0_hbm, o0_hbm), (p1_hbm, o1_hbm)):
            pltpu.emit_pipeline(
                body,
                grid=(NUM_TOKENS // SCW,),
                in_specs=[pl.BlockSpec((1, SCW), lambda i: (i, 0))],
                out_specs=[pl.BlockSpec((SCW, HU), lambda i: (i, 0))],
                core_axis_name=("c", "s"),
                dimension_semantics=(pltpu.PARALLEL,),
            )(p_hbm, o_hbm)
    return k(y_packed, p0_2d, p1_2d)


def _cast_kernel(x_ref, o_ref):
    o_ref[...] = x_ref[...].astype(jnp.bfloat16)


def _cast_bf16(x_sorted):
    return pl.pallas_call(
        _cast_kernel,
        grid=(NBLK,),
        in_specs=[pl.BlockSpec((BLK, HIDDEN), lambda i: (i, 0))],
        out_specs=pl.BlockSpec((BLK, HIDDEN), lambda i: (i, 0)),
        out_shape=jax.ShapeDtypeStruct((CAP, HIDDEN), jnp.bfloat16),
        compiler_params=pltpu.CompilerParams(
            dimension_semantics=("parallel",)),
    )(x_sorted)


def _ffn_ragged_kernel(cum_ref, x_ref, w1_hbm, w2_hbm, y_hbm,
                       w1_vmem, w2_vmem, wsem, ybuf, sems):
    e = pl.program_id(0)
    start = cum_ref[e]
    n = cum_ref[e + 1] - start
    cw1 = pltpu.make_async_copy(w1_hbm.at[e], w1_vmem, wsem.at[0])
    cw1.start()
    cw2 = pltpu.make_async_copy(w2_hbm.at[e], w2_vmem, wsem.at[1])
    cw2.start()
    cw1.wait()
    cw2.wait()
    w1 = w1_vmem[...]
    w2 = w2_vmem[...]

    def out_copy(i):
        base = (start + i) * BLK
        slot = jax.lax.rem(i, 2)
        return pltpu.make_async_copy(
            ybuf.at[slot], y_hbm.at[pl.ds(base, BLK)], sems.at[slot])

    def body(i, carry):
        base = (start + i) * BLK
        x = x_ref[pl.ds(base, BLK), :]
        slot = jax.lax.rem(i, 2)
        yacc = jnp.zeros((BLK, HIDDEN), jnp.float32)
        for f in range(4):
            w1f = w1[f * 1024:(f + 1) * 1024, :]
            h = jax.lax.dot_general(
                x, w1f, (((1,), (1,)), ((), ())),
                preferred_element_type=jnp.float32)
            g = 0.5 * h * (1.0 + jax.lax.erf(h * 0.7071067811865476))
            w2f = w2[:, f * 1024:(f + 1) * 1024]
            yacc = yacc + jax.lax.dot_general(
                g.astype(jnp.bfloat16), w2f, (((1,), (1,)), ((), ())),
                preferred_element_type=jnp.float32)

        @pl.when(i >= 2)
        def _():
            out_copy(i - 2).wait()

        ybuf[slot] = yacc.astype(jnp.bfloat16)
        out_copy(i).start()
        return carry

    jax.lax.fori_loop(0, n, body, 0)

    def drain(i, carry):
        out_copy(i).wait()
        return carry

    jax.lax.fori_loop(jnp.maximum(n - 2, 0), n, drain, 0)


def _ffn_ragged(xb_sorted, w1b, w2b, cum):
    grid_spec = pltpu.PrefetchScalarGridSpec(
        num_scalar_prefetch=1,
        grid=(NUM_EXPERTS,),
        in_specs=[
            pl.BlockSpec((CAP, HIDDEN), lambda e, cum_sm: (0, 0)),
            pl.BlockSpec(memory_space=pl.ANY),
            pl.BlockSpec(memory_space=pl.ANY),
        ],
        out_specs=pl.BlockSpec(memory_space=pl.ANY),
        scratch_shapes=[
            pltpu.VMEM((FFN, HIDDEN), jnp.bfloat16),
            pltpu.VMEM((HIDDEN, FFN), jnp.bfloat16),
            pltpu.SemaphoreType.DMA((2,)),
            pltpu.VMEM((2, BLK, HIDDEN), jnp.bfloat16),
            pltpu.SemaphoreType.DMA((2,)),
        ],
    )
    return pl.pallas_call(
        _ffn_ragged_kernel,
        grid_spec=grid_spec,
        out_shape=jax.ShapeDtypeStruct((CAP, HIDDEN), jnp.bfloat16),
        compiler_params=pltpu.CompilerParams(
            dimension_semantics=("arbitrary",)),
    )(cum, xb_sorted, w1b, w2b)


def _combine_kernel(y0_ref, y1_ref, am_ref, bm_ref, pvec_ref,
                    full_ref, fin_ref):
    y0 = y0_ref[...].astype(jnp.float32)
    y1 = y1_ref[...].astype(jnp.float32)
    am = am_ref[...]
    bm = bm_ref[...]
    p = pvec_ref[...]
    for e in range(NUM_EXPERTS):
        full_ref[:, e, :] = am[:, e][:, None] * y0 + bm[:, e][:, None] * y1
    p0 = jnp.sum(p * am, axis=1, keepdims=True)
    p1 = jnp.sum(p * bm, axis=1, keepdims=True)
    fin_ref[...] = p0 * y0 + p1 * y1


def _combine(y0, y1, am, bm, pvec):
    t, h = y0.shape
    e = am.shape[1]
    small = pl.BlockSpec((TB, e), lambda i: (i, 0))
    big = pl.BlockSpec((TB, h), lambda i: (i, 0))
    return pl.pallas_call(
        _combine_kernel,
        grid=(t // TB,),
        in_specs=[big, big, small, small, small],
        out_specs=(
            pl.BlockSpec((TB, e, h), lambda i: (i, 0, 0)),
            big,
        ),
        out_shape=(
            jax.ShapeDtypeStruct((t, e, h), jnp.float32),
            jax.ShapeDtypeStruct((t, h), jnp.float32),
        ),
        compiler_params=pltpu.CompilerParams(
            dimension_semantics=("parallel",)),
    )(y0, y1, am, bm, pvec)


@jax.jit
def kernel(hidden_states, gate_w, w1, w2):
    logits, am, bm, pvec = _router(hidden_states, gate_w)
    pos0, pos1, meta = _plan(am, bm)
    p0_2d = pos0.reshape(NUM_TOKENS // SCW, SCW)
    p1_2d = pos1.reshape(NUM_TOKENS // SCW, SCW)
    cum = meta.reshape(16)
    w1b = w1.astype(jnp.bfloat16)
    w2b = w2.astype(jnp.bfloat16)
    x_sorted = _sc_dispatch(hidden_states, p0_2d, p1_2d)
    xb_sorted = _cast_bf16(x_sorted)
    y_sorted = _ffn_ragged(xb_sorted, w1b, w2b, cum)
    y_packed = jax.lax.bitcast_convert_type(
        y_sorted.reshape(CAP, HU, 2), jnp.uint32)
    y0u, y1u = _sc_collect(y_packed, p0_2d, p1_2d)
    y0 = jax.lax.bitcast_convert_type(
        y0u, jnp.bfloat16).reshape(NUM_TOKENS, HIDDEN)
    y1 = jax.lax.bitcast_convert_type(
        y1u, jnp.bfloat16).reshape(NUM_TOKENS, HIDDEN)
    full, final = _combine(y0, y1, am, bm, pvec)
    return final, full, logits


# speedup vs baseline: 1.8805x; 1.8805x over previous
import jax
import jax.numpy as jnp
from jax.experimental import pallas as pl
from jax.experimental.pallas import tpu as pltpu
from jax.experimental.pallas import tpu_sc as plsc

NUM_TOKENS = 4096
HIDDEN = 1024
FFN = 4096
NUM_EXPERTS = 8
TOP_K = 2

TB = 256
BLK = 256
NBLK = (NUM_TOKENS * TOP_K) // BLK + NUM_EXPERTS
CAP = NBLK * BLK
CHUNK = 512
NCH = NUM_TOKENS // CHUNK
SCW = 16


def _router_kernel(x_ref, gw_ref, logits_ref, am_ref, bm_ref, pvec_ref):
    x = x_ref[...]
    gw = gw_ref[...]
    logits = jax.lax.dot_general(
        x, gw, (((1,), (1,)), ((), ())),
        preferred_element_type=jnp.float32,
        precision=jax.lax.Precision.DEFAULT)
    logits_ref[...] = logits
    e = gw.shape[0]
    iota = jax.lax.broadcasted_iota(jnp.int32, logits.shape, 1)
    m1 = jnp.max(logits, axis=1, keepdims=True)
    i1 = jnp.min(jnp.where(logits == m1, iota, e), axis=1, keepdims=True)
    a = iota == i1
    masked = jnp.where(a, -jnp.inf, logits)
    m2 = jnp.max(masked, axis=1, keepdims=True)
    i2 = jnp.min(jnp.where(masked == m2, iota, e), axis=1, keepdims=True)
    b = iota == i2
    q = jnp.exp(m2 - m1)
    p0 = 1.0 / (1.0 + q)
    p1 = q / (1.0 + q)
    af = a.astype(jnp.float32)
    bf = b.astype(jnp.float32)
    am_ref[...] = af
    bm_ref[...] = bf
    pvec_ref[...] = af * p0 + bf * p1


def _router(hidden_states, gate_w):
    t, _ = hidden_states.shape
    e = gate_w.shape[0]
    out_shapes = tuple(
        jax.ShapeDtypeStruct((t, e), jnp.float32) for _ in range(4))
    small = pl.BlockSpec((TB, e), lambda i: (i, 0))
    return pl.pallas_call(
        _router_kernel,
        grid=(t // TB,),
        in_specs=[
            pl.BlockSpec((TB, HIDDEN), lambda i: (i, 0)),
            pl.BlockSpec((e, HIDDEN), lambda i: (0, 0)),
        ],
        out_specs=(small, small, small, small),
        out_shape=out_shapes,
        compiler_params=pltpu.CompilerParams(
            dimension_semantics=("parallel",)),
    )(hidden_states, gate_w)


def _plan_kernel(am_ref, bm_ref, pos0_ref, pos1_ref, bmap_ref):
    am = am_ref[...]
    bm = bm_ref[...]
    s = am + bm
    ri = jax.lax.broadcasted_iota(jnp.int32, (CHUNK, CHUNK), 0)
    ci = jax.lax.broadcasted_iota(jnp.int32, (CHUNK, CHUNK), 1)
    lstrict = (ci < ri).astype(jnp.float32)
    off = jnp.zeros((1, NUM_EXPERTS), jnp.float32)
    pieces = []
    for c in range(NCH):
        sc = s[c * CHUNK:(c + 1) * CHUNK]
        p = jax.lax.dot_general(
            lstrict, sc, (((1,), (0,)), ((), ())),
            preferred_element_type=jnp.float32)
        pieces.append(p + off)
        off = off + jnp.sum(sc, axis=0, keepdims=True)
    prefix = jnp.concatenate(pieces, axis=0)
    counts = off
    nb = jnp.floor((counts + (BLK - 1.0)) * (1.0 / BLK))
    ei = jax.lax.broadcasted_iota(jnp.int32, (NUM_EXPERTS, NUM_EXPERTS), 0)
    ej = jax.lax.broadcasted_iota(jnp.int32, (NUM_EXPERTS, NUM_EXPERTS), 1)
    uincl = (ei <= ej).astype(jnp.float32)
    rbend = jax.lax.dot_general(
        nb, uincl, (((1,), (0,)), ((), ())),
        preferred_element_type=jnp.float32)
    base = float(BLK) * (rbend - nb)
    tgt = base + prefix
    pos0_ref[...] = jnp.sum(am * tgt, axis=1, keepdims=True).astype(jnp.int32)
    pos1_ref[...] = jnp.sum(bm * tgt, axis=1, keepdims=True).astype(jnp.int32)
    bi = jax.lax.broadcasted_iota(jnp.int32, (64, NUM_EXPERTS), 0)
    cmp = (bi >= rbend.astype(jnp.int32)).astype(jnp.float32)
    bmap = jnp.minimum(jnp.sum(cmp, axis=1, keepdims=True),
                       float(NUM_EXPERTS - 1))
    bi64 = jax.lax.broadcasted_iota(jnp.int32, (64, 1), 0)
    total = rbend[0, NUM_EXPERTS - 1]
    bmap = jnp.where(bi64 == 63, total, bmap)
    bmap_ref[...] = bmap.astype(jnp.int32)


def _plan(am, bm):
    t, e = am.shape
    return pl.pallas_call(
        _plan_kernel,
        grid=(1,),
        in_specs=[
            pl.BlockSpec((t, e), lambda i: (0, 0)),
            pl.BlockSpec((t, e), lambda i: (0, 0)),
        ],
        out_specs=(
            pl.BlockSpec((t, 1), lambda i: (0, 0)),
            pl.BlockSpec((t, 1), lambda i: (0, 0)),
            pl.BlockSpec((64, 1), lambda i: (0, 0)),
        ),
        out_shape=(
            jax.ShapeDtypeStruct((t, 1), jnp.int32),
            jax.ShapeDtypeStruct((t, 1), jnp.int32),
            jax.ShapeDtypeStruct((64, 1), jnp.int32),
        ),
    )(am, bm)


def _sc_mesh():
    return plsc.VectorSubcoreMesh(core_axis_name="c", subcore_axis_name="s")


def _sc_dispatch(xb, p0_2d, p1_2d):
    @pl.kernel(out_type=jax.ShapeDtypeStruct((CAP, HIDDEN), jnp.float32),
               mesh=_sc_mesh())
    def k(x_hbm, p0_hbm, p1_hbm, o_hbm):
        def body(x_vmem, i_vmem):
            pltpu.sync_copy(x_vmem, o_hbm.at[i_vmem.at[0]])
        for p_hbm in (p0_hbm, p1_hbm):
            pltpu.emit_pipeline(
                body,
                grid=(NUM_TOKENS // SCW,),
                in_specs=[
                    pl.BlockSpec((SCW, HIDDEN), lambda i: (i, 0)),
                    pl.BlockSpec((1, SCW), lambda i: (i, 0)),
                ],
                out_specs=[],
                core_axis_name=("c", "s"),
                dimension_semantics=(pltpu.PARALLEL,),
            )(x_hbm, p_hbm)
    return k(xb, p0_2d, p1_2d)


def _sc_collect(y_sorted, p0_2d, p1_2d):
    out_types = [jax.ShapeDtypeStruct((NUM_TOKENS, HIDDEN), jnp.float32)] * 2
    @pl.kernel(out_type=out_types, mesh=_sc_mesh())
    def k(y_hbm, p0_hbm, p1_hbm, o0_hbm, o1_hbm):
        def body(i_vmem, o_vmem):
            pltpu.sync_copy(y_hbm.at[i_vmem.at[0]], o_vmem)
        for p_hbm, o_hbm in ((p0_hbm, o0_hbm), (p1_hbm, o1_hbm)):
            pltpu.emit_pipeline(
                body,
                grid=(NUM_TOKENS // SCW,),
                in_specs=[pl.BlockSpec((1, SCW), lambda i: (i, 0))],
                out_specs=[pl.BlockSpec((SCW, HIDDEN), lambda i: (i, 0))],
                core_axis_name=("c", "s"),
                dimension_semantics=(pltpu.PARALLEL,),
            )(p_hbm, o_hbm)
    return k(y_sorted, p0_2d, p1_2d)


def _ffn_ragged_kernel(bmap_ref, x_ref, w1_ref, w2_ref, y_ref):
    nb = pl.program_id(0)
    total = bmap_ref[63]

    @pl.when(nb < total)
    def _():
        x = x_ref[...].astype(jnp.bfloat16)
        w1 = w1_ref[0]
        h = jax.lax.dot_general(
            x, w1, (((1,), (1,)), ((), ())),
            preferred_element_type=jnp.float32)
        g = 0.5 * h * (1.0 + jax.lax.erf(h * 0.7071067811865476))
        w2 = w2_ref[0]
        y_ref[...] = jax.lax.dot_general(
            g.astype(jnp.bfloat16), w2, (((1,), (1,)), ((), ())),
            preferred_element_type=jnp.float32)


def _ffn_ragged(x_sorted, w1b, w2b, bmap):
    grid_spec = pltpu.PrefetchScalarGridSpec(
        num_scalar_prefetch=1,
        grid=(NBLK,),
        in_specs=[
            pl.BlockSpec((BLK, HIDDEN), lambda nb, bmap_sm: (nb, 0)),
            pl.BlockSpec((1, FFN, HIDDEN),
                         lambda nb, bmap_sm: (bmap_sm[nb], 0, 0)),
            pl.BlockSpec((1, HIDDEN, FFN),
                         lambda nb, bmap_sm: (bmap_sm[nb], 0, 0)),
        ],
        out_specs=pl.BlockSpec((BLK, HIDDEN), lambda nb, bmap_sm: (nb, 0)),
    )
    return pl.pallas_call(
        _ffn_ragged_kernel,
        grid_spec=grid_spec,
        out_shape=jax.ShapeDtypeStruct((CAP, HIDDEN), jnp.float32),
        compiler_params=pltpu.CompilerParams(
            dimension_semantics=("parallel",)),
    )(bmap, x_sorted, w1b, w2b)


def _combine_kernel(y0_ref, y1_ref, am_ref, bm_ref, pvec_ref,
                    full_ref, fin_ref):
    y0 = y0_ref[...]
    y1 = y1_ref[...]
    am = am_ref[...]
    bm = bm_ref[...]
    p = pvec_ref[...]
    for e in range(NUM_EXPERTS):
        full_ref[:, e, :] = am[:, e][:, None] * y0 + bm[:, e][:, None] * y1
    p0 = jnp.sum(p * am, axis=1, keepdims=True)
    p1 = jnp.sum(p * bm, axis=1, keepdims=True)
    fin_ref[...] = p0 * y0 + p1 * y1


def _combine(y0, y1, am, bm, pvec):
    t, h = y0.shape
    e = am.shape[1]
    small = pl.BlockSpec((TB, e), lambda i: (i, 0))
    big = pl.BlockSpec((TB, h), lambda i: (i, 0))
    return pl.pallas_call(
        _combine_kernel,
        grid=(t // TB,),
        in_specs=[big, big, small, small, small],
        out_specs=(
            pl.BlockSpec((TB, e, h), lambda i: (i, 0, 0)),
            big,
        ),
        out_shape=(
            jax.ShapeDtypeStruct((t, e, h), jnp.float32),
            jax.ShapeDtypeStruct((t, h), jnp.float32),
        ),
        compiler_params=pltpu.CompilerParams(
            dimension_semantics=("parallel",)),
    )(y0, y1, am, bm, pvec)


@jax.jit
def kernel(hidden_states, gate_w, w1, w2):
    logits, am, bm, pvec = _router(hidden_states, gate_w)
    pos0, pos1, bmap = _plan(am, bm)
    p0_2d = pos0.reshape(NUM_TOKENS // SCW, SCW)
    p1_2d = pos1.reshape(NUM_TOKENS // SCW, SCW)
    bmap_1d = bmap.reshape(64)
    w1b = w1.astype(jnp.bfloat16)
    w2b = w2.astype(jnp.bfloat16)
    x_sorted = _sc_dispatch(hidden_states, p0_2d, p1_2d)
    y_sorted = _ffn_ragged(x_sorted, w1b, w2b, bmap_1d)
    y0, y1 = _sc_collect(y_sorted, p0_2d, p1_2d)
    full, final = _combine(y0, y1, am, bm, pvec)
    return final, full, logits
